# baseline (speedup 1.0000x reference)
import functools

import jax
import jax.numpy as jnp
from jax import lax
from jax.experimental import pallas as pl
from jax.experimental.pallas import tpu as pltpu
from jax.experimental.pallas import tpu_sc as plsc

_T_AGENT_OF_INTEREST = 0
_T_AGENT_NO = 1
_T_AGENT_CAR = 2
_T_ROUTE = 3
_T_LANE_CENTER = 4
_T_BOUND = 5

_NUM_CORES = 2
_NUM_SUBCORES = 16
_NW = _NUM_CORES * _NUM_SUBCORES
_LANES = 16

_SC_FRACTION = 0.25
_TC_BLOCK = 32


@functools.lru_cache(maxsize=None)
def _build_sc_call(batch, total_len, dim, segments):
    bpw = batch // _NW
    n_lane_chunks = dim // _LANES
    mesh = plsc.VectorSubcoreMesh(core_axis_name="c", subcore_axis_name="s")

    def body(emb_hbm, out_hbm, emb_v, pat_v, sem):
        cid = lax.axis_index("c")
        sid = lax.axis_index("s")
        wid = sid * _NUM_CORES + cid

        pltpu.sync_copy(emb_hbm, emb_v)

        for start, seg_len, t in segments:
            row = [emb_v[t, pl.ds(j * _LANES, _LANES)]
                   for j in range(n_lane_chunks)]
            if seg_len == 1:
                for j in range(n_lane_chunks):
                    pat_v[start, pl.ds(j * _LANES, _LANES)] = row[j]
            else:
                def fill(i, _, start=start, row=row):
                    for j in range(n_lane_chunks):
                        pat_v[start + i, pl.ds(j * _LANES, _LANES)] = row[j]
                    return 0
                lax.fori_loop(0, seg_len, fill, 0, unroll=4)

        @pl.when(cid == 0)
        def _stream():
            copies = [pltpu.async_copy(
                pat_v, out_hbm.at[sid + i * _NUM_SUBCORES], sem)
                for i in range(batch // _NUM_SUBCORES)]
            for cp in copies:
                cp.wait()

    return pl.kernel(
        body,
        out_type=jax.ShapeDtypeStruct((batch, total_len, dim), jnp.float32),
        mesh=mesh,
        scratch_types=[
            pltpu.VMEM((6, dim), jnp.float32),
            pltpu.VMEM((total_len, dim), jnp.float32),
            pltpu.SemaphoreType.DMA,
        ],
    )


@functools.lru_cache(maxsize=None)
def _build_tc_call(batch, total_len, dim, segments):
    bb = min(_TC_BLOCK, batch)
    assert batch % bb == 0

    def body(emb_ref, out_ref):
        parts = [jnp.broadcast_to(emb_ref[t:t + 1, :], (seg_len, dim))
                 for _, seg_len, t in segments]
        rows = jnp.concatenate(parts, axis=0)
        out_ref[...] = jnp.broadcast_to(rows[None], (bb, total_len, dim))

    return pl.pallas_call(
        body,
        grid=(batch // bb,),
        in_specs=[pl.BlockSpec((6, dim), lambda i: (0, 0))],
        out_specs=pl.BlockSpec((bb, total_len, dim), lambda i: (i, 0, 0)),
        out_shape=jax.ShapeDtypeStruct((batch, total_len, dim), jnp.float32),
    )


def kernel(ego, obs, lane, bound, embedding):
    batch = ego.shape[0]
    other_agents_len = obs.shape[1]
    route_len = 1
    lanes_len = lane.shape[1]
    bounds_len = bound.shape[1]
    total_len = 1 + other_agents_len + route_len + lanes_len + bounds_len
    dim = embedding.shape[1]

    other_start = 1
    route_start = other_start + other_agents_len
    lanes_start = route_start + route_len
    bounds_start = lanes_start + lanes_len
    segments = (
        (0, 1, _T_AGENT_OF_INTEREST),
        (other_start, other_agents_len, _T_AGENT_CAR),
        (route_start, route_len, _T_ROUTE),
        (lanes_start, lanes_len, _T_LANE_CENTER),
        (bounds_start, bounds_len, _T_BOUND),
    )

    return _build_sc_call(batch, total_len, dim, segments)(embedding)


# speedup vs baseline: 1.2193x; 1.2193x over previous
import functools

import jax
import jax.numpy as jnp
from jax import lax
from jax.experimental import pallas as pl
from jax.experimental.pallas import tpu as pltpu
from jax.experimental.pallas import tpu_sc as plsc

_T_AGENT_OF_INTEREST = 0
_T_AGENT_NO = 1
_T_AGENT_CAR = 2
_T_ROUTE = 3
_T_LANE_CENTER = 4
_T_BOUND = 5

_NUM_CORES = 2
_NUM_SUBCORES = 16
_NW = _NUM_CORES * _NUM_SUBCORES
_LANES = 16

_SC_FRACTION = 0.5
_TC_BLOCK = 32


@functools.lru_cache(maxsize=None)
def _build_sc_call(batch, sc_batch, total_len, dim, segments):
    bpw = sc_batch // _NW
    n_lane_chunks = dim // _LANES
    mesh = plsc.VectorSubcoreMesh(core_axis_name="c", subcore_axis_name="s")

    def body(emb_hbm, out_hbm, emb_v, pat_v, sem):
        cid = lax.axis_index("c")
        sid = lax.axis_index("s")
        wid = sid * _NUM_CORES + cid

        pltpu.sync_copy(emb_hbm, emb_v)

        for start, seg_len, t in segments:
            row = [emb_v[t, pl.ds(j * _LANES, _LANES)]
                   for j in range(n_lane_chunks)]
            if seg_len == 1:
                for j in range(n_lane_chunks):
                    pat_v[start, pl.ds(j * _LANES, _LANES)] = row[j]
            else:
                def fill(i, _, start=start, row=row):
                    for j in range(n_lane_chunks):
                        pat_v[start + i, pl.ds(j * _LANES, _LANES)] = row[j]
                    return 0
                lax.fori_loop(0, seg_len, fill, 0, unroll=4)

        copies = [pltpu.async_copy(pat_v, out_hbm.at[wid + i * _NW], sem)
                  for i in range(bpw)]
        for cp in copies:
            cp.wait()

    return pl.kernel(
        body,
        out_type=jax.ShapeDtypeStruct((batch, total_len, dim), jnp.float32),
        mesh=mesh,
        scratch_types=[
            pltpu.VMEM((6, dim), jnp.float32),
            pltpu.VMEM((total_len, dim), jnp.float32),
            pltpu.SemaphoreType.DMA,
        ],
    )


@functools.lru_cache(maxsize=None)
def _build_tc_call(batch, tc_start, total_len, dim, segments):
    bb = _TC_BLOCK
    tc_batch = batch - tc_start
    assert tc_batch % bb == 0 and tc_start % bb == 0
    base_blk = tc_start // bb

    def body(_, emb_ref, out_ref):
        parts = [jnp.broadcast_to(emb_ref[t:t + 1, :], (seg_len, dim))
                 for _, seg_len, t in segments]
        rows = jnp.concatenate(parts, axis=0)
        out_ref[...] = jnp.broadcast_to(rows[None], (bb, total_len, dim))

    return pl.pallas_call(
        body,
        grid=(tc_batch // bb,),
        in_specs=[
            pl.BlockSpec(memory_space=pl.ANY),
            pl.BlockSpec((6, dim), lambda i: (0, 0)),
        ],
        out_specs=pl.BlockSpec((bb, total_len, dim),
                               lambda i: (base_blk + i, 0, 0)),
        out_shape=jax.ShapeDtypeStruct((batch, total_len, dim), jnp.float32),
        input_output_aliases={0: 0},
    )


def kernel(ego, obs, lane, bound, embedding):
    batch = ego.shape[0]
    other_agents_len = obs.shape[1]
    route_len = 1
    lanes_len = lane.shape[1]
    bounds_len = bound.shape[1]
    total_len = 1 + other_agents_len + route_len + lanes_len + bounds_len
    dim = embedding.shape[1]

    other_start = 1
    route_start = other_start + other_agents_len
    lanes_start = route_start + route_len
    bounds_start = lanes_start + lanes_len
    segments = (
        (0, 1, _T_AGENT_OF_INTEREST),
        (other_start, other_agents_len, _T_AGENT_CAR),
        (route_start, route_len, _T_ROUTE),
        (lanes_start, lanes_len, _T_LANE_CENTER),
        (bounds_start, bounds_len, _T_BOUND),
    )

    sc_batch = int(batch * _SC_FRACTION) // _NW * _NW
    sc_out = _build_sc_call(batch, sc_batch, total_len, dim, segments)(
        embedding)
    if sc_batch == batch:
        return sc_out
    return _build_tc_call(batch, sc_batch, total_len, dim, segments)(
        sc_out, embedding)


# speedup vs baseline: 1.2381x; 1.0154x over previous
import functools

import jax
import jax.numpy as jnp
from jax import lax
from jax.experimental import pallas as pl
from jax.experimental.pallas import tpu as pltpu
from jax.experimental.pallas import tpu_sc as plsc

_T_AGENT_OF_INTEREST = 0
_T_AGENT_NO = 1
_T_AGENT_CAR = 2
_T_ROUTE = 3
_T_LANE_CENTER = 4
_T_BOUND = 5

_NUM_CORES = 2
_NUM_SUBCORES = 16
_NW = _NUM_CORES * _NUM_SUBCORES
_LANES = 16


@functools.lru_cache(maxsize=None)
def _build_sc_call(batch, total_len, dim, segments):
    bpw = batch // _NW
    n_lane_chunks = dim // _LANES
    mesh = plsc.VectorSubcoreMesh(core_axis_name="c", subcore_axis_name="s")

    def body(emb_hbm, out_hbm, emb_v, pat_v, sem):
        cid = lax.axis_index("c")
        sid = lax.axis_index("s")
        wid = sid * _NUM_CORES + cid

        pltpu.sync_copy(emb_hbm, emb_v)

        for start, seg_len, t in segments:
            row = [emb_v[t, pl.ds(j * _LANES, _LANES)]
                   for j in range(n_lane_chunks)]
            if seg_len == 1:
                for j in range(n_lane_chunks):
                    pat_v[start, pl.ds(j * _LANES, _LANES)] = row[j]
            else:
                def fill(i, _, start=start, row=row):
                    for j in range(n_lane_chunks):
                        pat_v[start + i, pl.ds(j * _LANES, _LANES)] = row[j]
                    return 0
                lax.fori_loop(0, seg_len, fill, 0, unroll=4)

        copies = [pltpu.async_copy(pat_v, out_hbm.at[wid + i * _NW], sem)
                  for i in range(bpw)]
        for cp in copies:
            cp.wait()

    return pl.kernel(
        body,
        out_type=jax.ShapeDtypeStruct((batch, total_len, dim), jnp.float32),
        mesh=mesh,
        scratch_types=[
            pltpu.VMEM((6, dim), jnp.float32),
            pltpu.VMEM((total_len, dim), jnp.float32),
            pltpu.SemaphoreType.DMA,
        ],
    )


def kernel(ego, obs, lane, bound, embedding):
    batch = ego.shape[0]
    other_agents_len = obs.shape[1]
    route_len = 1
    lanes_len = lane.shape[1]
    bounds_len = bound.shape[1]
    total_len = 1 + other_agents_len + route_len + lanes_len + bounds_len
    dim = embedding.shape[1]

    other_start = 1
    route_start = other_start + other_agents_len
    lanes_start = route_start + route_len
    bounds_start = lanes_start + lanes_len
    segments = (
        (0, 1, _T_AGENT_OF_INTEREST),
        (other_start, other_agents_len, _T_AGENT_CAR),
        (route_start, route_len, _T_ROUTE),
        (lanes_start, lanes_len, _T_LANE_CENTER),
        (bounds_start, bounds_len, _T_BOUND),
    )

    return _build_sc_call(batch, total_len, dim, segments)(embedding)
